# baseline (speedup 1.0000x reference)
import functools

import jax
import jax.numpy as jnp
from jax import lax
from jax.experimental import pallas as pl
from jax.experimental.pallas import tpu as pltpu
from jax.experimental.pallas import tpu_sc as plsc

NBUF = 4
WPC = 4


def _make_lookup(batch, word_len, vocab, dim, num_workers):
    chunk = WPC * word_len
    assert chunk <= 128
    assert batch % (num_workers * WPC) == 0
    words_per_worker = batch // num_workers
    chunks = words_per_worker // WPC
    rounds = chunks // NBUF
    assert chunks % NBUF == 0

    mesh = plsc.VectorSubcoreMesh(core_axis_name="c", subcore_axis_name="s")

    @functools.partial(
        pl.kernel,
        mesh=mesh,
        out_type=jax.ShapeDtypeStruct((batch, word_len, dim), jnp.float32),
        compiler_params=pltpu.CompilerParams(use_tc_tiling_on_sc=True),
        scratch_types=(
            [pltpu.VMEM((chunks, chunk), jnp.int32)]
            + [pltpu.VMEM((chunk, dim), jnp.float32) for _ in range(NBUF)]
            + [pltpu.SemaphoreType.DMA for _ in range(2 * NBUF)]
        ),
    )
    def lookup(idx_hbm, table_hbm, out_hbm, idx_v, *scratch):
        bufs = scratch[:NBUF]
        gsems = scratch[NBUF : 2 * NBUF]
        ssems = scratch[2 * NBUF :]
        nc = lax.axis_size("c")
        wid = lax.axis_index("s") * nc + lax.axis_index("c")
        pltpu.sync_copy(idx_hbm.at[pl.ds(wid * chunks, chunks)], idx_v)
        word_base = wid * words_per_worker

        def gather(b, j):
            pltpu.async_copy(table_hbm.at[idx_v.at[j]], bufs[b], gsems[b])

        def stores(b, j):
            pltpu.async_copy(
                bufs[b].reshape(WPC, word_len, dim),
                out_hbm.at[pl.ds(word_base + j * WPC, WPC)],
                ssems[b],
            )

        def wait_stores(b):
            pltpu.make_async_copy(
                bufs[b].reshape(WPC, word_len, dim),
                out_hbm.at[pl.ds(0, WPC)],
                ssems[b],
            ).wait()

        for b in range(NBUF):
            gather(b, b)

        def round_step(r, carry):
            for b in range(NBUF):
                j = r * NBUF + b
                pltpu.make_async_copy(table_hbm.at[idx_v.at[j]], bufs[b], gsems[b]).wait()
                stores(b, j)

                @pl.when(j + NBUF < chunks)
                def _():
                    wait_stores(b)
                    gather(b, j + NBUF)

            return carry

        lax.fori_loop(0, rounds, round_step, 0)
        for b in range(NBUF):
            wait_stores(b)

    return lookup


NPART = 2


def kernel(char_sequences, char_emb_table):
    batch, word_len = char_sequences.shape
    vocab, dim = char_emb_table.shape
    info = plsc.get_sparse_core_info()
    num_workers = info.num_cores * info.num_subcores
    part = batch // NPART
    lookup = _make_lookup(part, word_len, vocab, dim, num_workers)
    outs = []
    for h in range(NPART):
        idx2d = char_sequences[h * part : (h + 1) * part].reshape(part // WPC, WPC * word_len)
        outs.append(lookup(idx2d, char_emb_table))
    return jnp.concatenate(outs, axis=0)


# speedup vs baseline: 2.1904x; 2.1904x over previous
import functools

import jax
import jax.numpy as jnp
from jax import lax
from jax.experimental import pallas as pl
from jax.experimental.pallas import tpu as pltpu
from jax.experimental.pallas import tpu_sc as plsc

NBUF = 4
WPC = 4


def _make_lookup(batch, word_len, vocab, dim, num_workers):
    chunk = WPC * word_len
    assert chunk <= 128
    assert batch % (num_workers * WPC) == 0
    words_per_worker = batch // num_workers
    chunks = words_per_worker // WPC
    rounds = chunks // NBUF
    assert chunks % NBUF == 0

    mesh = plsc.VectorSubcoreMesh(core_axis_name="c", subcore_axis_name="s")

    @functools.partial(
        pl.kernel,
        mesh=mesh,
        out_type=jax.ShapeDtypeStruct((batch, word_len, dim), jnp.float32),
        scratch_types=(
            [
                pltpu.VMEM_SHARED((vocab, dim), jnp.float32),
                pltpu.VMEM((chunks, chunk), jnp.int32),
            ]
            + [pltpu.VMEM((chunk, dim), jnp.float32) for _ in range(NBUF)]
            + [pltpu.SemaphoreType.DMA for _ in range(2 * NBUF)]
        ),
    )
    def lookup(idx_hbm, table_hbm, out_hbm, table_sp, idx_v, *scratch):
        bufs = scratch[:NBUF]
        gsems = scratch[NBUF : 2 * NBUF]
        ssems = scratch[2 * NBUF :]
        nc = lax.axis_size("c")
        sid = lax.axis_index("s")
        wid = sid * nc + lax.axis_index("c")

        @pl.when(sid == 0)
        def _():
            pltpu.sync_copy(table_hbm, table_sp)

        pltpu.sync_copy(idx_hbm.at[pl.ds(wid * chunks, chunks)], idx_v)
        plsc.subcore_barrier()
        word_base = wid * words_per_worker

        def gather(b, j):
            pltpu.async_copy(table_sp.at[idx_v.at[j]], bufs[b], gsems[b])

        def stores(b, j):
            pltpu.async_copy(
                bufs[b].reshape(WPC, word_len, dim),
                out_hbm.at[pl.ds(word_base + j * WPC, WPC)],
                ssems[b],
            )

        def wait_stores(b):
            pltpu.make_async_copy(
                bufs[b].reshape(WPC, word_len, dim),
                out_hbm.at[pl.ds(0, WPC)],
                ssems[b],
            ).wait()

        for b in range(NBUF):
            gather(b, b)

        def round_step(r, carry):
            for b in range(NBUF):
                j = r * NBUF + b
                pltpu.make_async_copy(table_sp.at[idx_v.at[j]], bufs[b], gsems[b]).wait()
                stores(b, j)

                @pl.when(j + NBUF < chunks)
                def _():
                    wait_stores(b)
                    gather(b, j + NBUF)

            return carry

        lax.fori_loop(0, rounds, round_step, 0)
        for b in range(NBUF):
            wait_stores(b)

    return lookup


def kernel(char_sequences, char_emb_table):
    batch, word_len = char_sequences.shape
    vocab, dim = char_emb_table.shape
    idx2d = char_sequences.reshape(batch // WPC, WPC * word_len)
    info = plsc.get_sparse_core_info()
    num_workers = info.num_cores * info.num_subcores
    return _make_lookup(batch, word_len, vocab, dim, num_workers)(idx2d, char_emb_table)
